# baseline (speedup 1.0000x reference)
import jax
import jax.numpy as jnp
from jax import lax
from jax.experimental import pallas as pl
from jax.experimental.pallas import tpu as pltpu
from jax.experimental.pallas import tpu_sc as plsc

_N = 10000
_E = 320000
_D = 128
_CHUNK = 128
_NC = 2
_NS = 16
_NW = _NC * _NS
_EROWS = -(-_E // _CHUNK)
_EROWS_PAD = -(-_EROWS // (_NW * 8)) * _NW * 8
_RPW = _EROWS_PAD // _NS
_ACC_ROWS = 10240
_ZROWS = _ACC_ROWS // _NS
_WROWS = 624
_WLAST = _N - 15 * _WROWS

_SC_PARAMS = pltpu.CompilerParams(use_tc_tiling_on_sc=False)
_MESH = plsc.VectorSubcoreMesh(core_axis_name="c", subcore_axis_name="s")
_MESH1 = plsc.VectorSubcoreMesh(core_axis_name="c", subcore_axis_name="s",
                                num_cores=1, num_subcores=_NS)


def _agg_body(h_hbm, srcr, dstr, sums_hbm, idx_s, idx_d, rows2, acc,
              sem0, sem1):
  cid = lax.axis_index("c")
  sid = lax.axis_index("s")
  sems = (sem0, sem1)

  @pl.when(cid == 0)
  def _():
    @pl.loop(0, _CHUNK)
    def _(r):
      for c in range(_D // 16):
        rows2[0, r, pl.ds(c * 16, 16)] = jnp.zeros((16,), jnp.float32)

    z0 = sid * _ZROWS
    for k in range(_ZROWS // _CHUNK):
      pltpu.sync_copy(rows2.at[0], acc.at[pl.ds(z0 + k * _CHUNK, _CHUNK)])

  plsc.subcore_barrier()

  def stage(r0, c):
    pltpu.sync_copy(srcr.at[pl.ds(r0 + c * 8, 8)], idx_s)
    pltpu.sync_copy(dstr.at[pl.ds(r0 + c * 8, 8)], idx_d)

  def wait_scatter(b):
    pltpu.make_async_copy(rows2.at[b], acc.at[idx_d.at[b]], sems[b]).wait()

  def row(j, wait):
    b = j % 2
    if wait:
      wait_scatter(b)
    pltpu.sync_copy(h_hbm.at[idx_s.at[j]], rows2.at[b])
    pltpu.async_copy(rows2.at[b], acc.at[idx_d.at[j]], sems[b], add=True)

  def pipeline(r0, nchunk):
    stage(r0, 0)
    for j in range(8):
      row(j, j >= 2)

    @pl.loop(1, nchunk)
    def _(c):
      wait_scatter(0)
      wait_scatter(1)
      stage(r0, c)
      for j in range(8):
        row(j, j >= 2)

    wait_scatter(0)
    wait_scatter(1)

  @pl.when(cid == 0)
  def _():
    pipeline(sid * _RPW, _RPW // 16)
    pipeline(sid * _RPW + _RPW // 2, _RPW // 16)

  plsc.subcore_barrier()

  @pl.when(cid == 0)
  def _():
    w0 = sid * _WROWS

    @pl.when(sid < _NS - 1)
    def _():
      pltpu.sync_copy(acc.at[pl.ds(w0, _WROWS)], sums_hbm.at[pl.ds(w0, _WROWS)])

    @pl.when(sid == _NS - 1)
    def _():
      w1 = (_NS - 1) * _WROWS
      pltpu.sync_copy(acc.at[pl.ds(w1, _WLAST)], sums_hbm.at[pl.ds(w1, _WLAST)])


_sc_agg = pl.kernel(
    _agg_body,
    out_type=jax.ShapeDtypeStruct((_N, _D), jnp.float32),
    mesh=_MESH,
    scratch_types=[
        pltpu.VMEM((8, _CHUNK), jnp.int32),
        pltpu.VMEM((8, _CHUNK), jnp.int32),
        pltpu.VMEM((2, _CHUNK, _D), jnp.float32),
        pltpu.VMEM_SHARED((_ACC_ROWS, _D), jnp.float32),
        pltpu.SemaphoreType.DMA,
        pltpu.SemaphoreType.DMA,
    ],
    compiler_params=_SC_PARAMS)


def _count_body(dstr, z16, ones_h, cnt_hbm, idx_d, ones_v, cacc, csem):
  sid = lax.axis_index("s")

  z0 = sid * _ZROWS
  pltpu.sync_copy(z16.at[pl.ds(0, _ZROWS)], cacc.at[pl.ds(z0, _ZROWS)])
  pltpu.sync_copy(ones_h, ones_v)
  plsc.subcore_barrier()

  pltpu.sync_copy(dstr.at[pl.ds(sid * _RPW, _RPW)], idx_d)

  @pl.loop(0, _RPW // 8)
  def _(c):
    for j in range(8):
      pltpu.async_copy(ones_v, cacc.at[idx_d.at[c * 8 + j]], csem, add=True)
    for j in range(8):
      pltpu.make_async_copy(ones_v, cacc.at[idx_d.at[c * 8 + j]], csem).wait()

  plsc.subcore_barrier()
  w0 = sid * _WROWS

  @pl.when(sid < _NS - 1)
  def _():
    pltpu.sync_copy(cacc.at[pl.ds(w0, _WROWS)], cnt_hbm.at[pl.ds(w0, _WROWS)])

  @pl.when(sid == _NS - 1)
  def _():
    w1 = (_NS - 1) * _WROWS
    pltpu.sync_copy(cacc.at[pl.ds(w1, _WLAST)], cnt_hbm.at[pl.ds(w1, _WLAST)])


_sc_count = pl.kernel(
    _count_body,
    out_type=jax.ShapeDtypeStruct((_N, 16), jnp.float32),
    mesh=_MESH1,
    scratch_types=[
        pltpu.VMEM((_RPW, _CHUNK), jnp.int32),
        pltpu.VMEM((_CHUNK, 16), jnp.float32),
        pltpu.VMEM_SHARED((_ACC_ROWS, 16), jnp.float32),
        pltpu.SemaphoreType.DMA,
    ],
    compiler_params=_SC_PARAMS)


_R = 1000


def _self_mm_body(h_r, w_r, o_r):
  o_r[...] = jnp.dot(h_r[...], w_r[...], preferred_element_type=jnp.float32)


def _combine_body(p_r, s_r, c_r, wn_r, b_r, o_r):
  inv = 1.0 / jnp.maximum(c_r[:, 0:1], 1.0)
  mean = s_r[...] * inv
  h = (p_r[...]
       + jnp.dot(mean, wn_r[...], preferred_element_type=jnp.float32)
       + b_r[...])
  o_r[...] = jnp.maximum(h, 0.0)


def _combine2_body(p_r, s_r, c_r, wn_r, b_r, wc1_r, bc1_r, wc2_r, bc2_r, o_r):
  inv = 1.0 / jnp.maximum(c_r[:, 0:1], 1.0)
  mean = s_r[...] * inv
  h2 = (p_r[...]
        + jnp.dot(mean, wn_r[...], preferred_element_type=jnp.float32)
        + b_r[...])
  h2 = jnp.maximum(h2, 0.0)
  hid = jnp.maximum(
      jnp.dot(h2, wc1_r[...], preferred_element_type=jnp.float32) + bc1_r[...],
      0.0)
  o_r[...] = (jnp.dot(hid, wc2_r[...], preferred_element_type=jnp.float32)
              + bc2_r[...])


def _row_spec(w):
  return pl.BlockSpec((_R, w), lambda i: (i, 0))


def _full_spec(h, w):
  return pl.BlockSpec((h, w), lambda i: (0, 0))


def _self_mm(h, W):
  return pl.pallas_call(
      _self_mm_body,
      grid=(_N // _R,),
      in_specs=[_row_spec(_D), _full_spec(_D, _D)],
      out_specs=_row_spec(_D),
      out_shape=jax.ShapeDtypeStruct((_N, _D), jnp.float32),
  )(h, W)


def _combine1(p, sums, cnts, Wn, b):
  return pl.pallas_call(
      _combine_body,
      grid=(_N // _R,),
      in_specs=[
          _row_spec(_D), _row_spec(_D), _row_spec(16),
          _full_spec(_D, _D), _full_spec(1, _D),
      ],
      out_specs=_row_spec(_D),
      out_shape=jax.ShapeDtypeStruct((_N, _D), jnp.float32),
  )(p, sums, cnts, Wn, b.reshape(1, _D))


def _combine2(p, sums, cnts, Wn, b, Wc1, bc1, Wc2, bc2):
  n_cls = Wc2.shape[1]
  cls_hid = Wc1.shape[1]
  return pl.pallas_call(
      _combine2_body,
      grid=(_N // _R,),
      in_specs=[
          _row_spec(_D), _row_spec(_D), _row_spec(16),
          _full_spec(_D, _D), _full_spec(1, _D),
          _full_spec(_D, cls_hid), _full_spec(1, cls_hid),
          _full_spec(cls_hid, n_cls), _full_spec(1, n_cls),
      ],
      out_specs=_row_spec(n_cls),
      out_shape=jax.ShapeDtypeStruct((_N, n_cls), jnp.float32),
  )(p, sums, cnts, Wn, b.reshape(1, _D), Wc1, bc1.reshape(1, cls_hid),
    Wc2, bc2.reshape(1, n_cls))


def kernel(x, edge_index, W1_self, W1_neigh, b1, W2_self, W2_neigh, b2,
           Wc1, bc1, Wc2, bc2):
  pad = _EROWS_PAD * _CHUNK - _E
  src = jnp.concatenate([edge_index[0], jnp.zeros((pad,), jnp.int32)])
  dst = jnp.concatenate([edge_index[1], jnp.full((pad,), _N, jnp.int32)])
  src = src.reshape(_EROWS_PAD, _CHUNK)
  dst = dst.reshape(_EROWS_PAD, _CHUNK)

  z16 = jnp.zeros((_ZROWS, 16), jnp.float32)
  ones16 = jnp.ones((_CHUNK, 16), jnp.float32)

  cnts = _sc_count(dst, z16, ones16)
  sums1 = _sc_agg(x, src, dst)
  p1 = _self_mm(x, W1_self)
  h1 = _combine1(p1, sums1, cnts, W1_neigh, b1)

  sums2 = _sc_agg(h1, src, dst)
  p2 = _self_mm(h1, W2_self)
  return _combine2(p2, sums2, cnts, W2_neigh, b2, Wc1, bc1, Wc2, bc2)


# speedup vs baseline: 1.0092x; 1.0092x over previous
import jax
import jax.numpy as jnp
from jax import lax
from jax.experimental import pallas as pl
from jax.experimental.pallas import tpu as pltpu
from jax.experimental.pallas import tpu_sc as plsc

_N = 10000
_E = 320000
_D = 128
_CHUNK = 128
_NC = 2
_NS = 16
_NW = _NC * _NS
_EROWS = -(-_E // _CHUNK)
_EROWS_PAD = -(-_EROWS // (_NW * 8)) * _NW * 8
_RPW = _EROWS_PAD // _NS
_ACC_ROWS = 10240
_ZROWS = _ACC_ROWS // _NS
_WROWS = 624
_WLAST = _N - 15 * _WROWS

_SC_PARAMS = pltpu.CompilerParams(use_tc_tiling_on_sc=False)
_MESH = plsc.VectorSubcoreMesh(core_axis_name="c", subcore_axis_name="s")
_MESH1 = plsc.VectorSubcoreMesh(core_axis_name="c", subcore_axis_name="s",
                                num_cores=1, num_subcores=_NS)


def _agg_body(h_hbm, srcr, dstr, sums_hbm, idx_s, idx_d, rows2, acc,
              sem0, sem1):
  cid = lax.axis_index("c")
  sid = lax.axis_index("s")
  sems = (sem0, sem1)

  @pl.when(cid == 0)
  def _():
    @pl.loop(0, _CHUNK)
    def _(r):
      for c in range(_D // 16):
        rows2[0, r, pl.ds(c * 16, 16)] = jnp.zeros((16,), jnp.float32)

    z0 = sid * _ZROWS
    for k in range(_ZROWS // _CHUNK):
      pltpu.sync_copy(rows2.at[0], acc.at[pl.ds(z0 + k * _CHUNK, _CHUNK)])

  plsc.subcore_barrier()

  def stage(r0, c):
    pltpu.sync_copy(srcr.at[pl.ds(r0 + c * 8, 8)], idx_s)
    pltpu.sync_copy(dstr.at[pl.ds(r0 + c * 8, 8)], idx_d)

  def wait_scatter(b):
    pltpu.make_async_copy(rows2.at[b], acc.at[idx_d.at[b]], sems[b]).wait()

  def row(j, wait):
    b = j % 2
    if wait:
      wait_scatter(b)
    pltpu.sync_copy(h_hbm.at[idx_s.at[j]], rows2.at[b])
    pltpu.async_copy(rows2.at[b], acc.at[idx_d.at[j]], sems[b], add=True)

  def pipeline(r0, nchunk):
    stage(r0, 0)
    for j in range(8):
      row(j, j >= 2)

    @pl.loop(1, nchunk)
    def _(c):
      wait_scatter(0)
      wait_scatter(1)
      stage(r0, c)
      for j in range(8):
        row(j, j >= 2)

    wait_scatter(0)
    wait_scatter(1)

  @pl.when(cid == 0)
  def _():
    pipeline(sid * _RPW, _RPW // 8)

  plsc.subcore_barrier()

  @pl.when(cid == 0)
  def _():
    w0 = sid * _WROWS

    @pl.when(sid < _NS - 1)
    def _():
      pltpu.sync_copy(acc.at[pl.ds(w0, _WROWS)], sums_hbm.at[pl.ds(w0, _WROWS)])

    @pl.when(sid == _NS - 1)
    def _():
      w1 = (_NS - 1) * _WROWS
      pltpu.sync_copy(acc.at[pl.ds(w1, _WLAST)], sums_hbm.at[pl.ds(w1, _WLAST)])


_sc_agg = pl.kernel(
    _agg_body,
    out_type=jax.ShapeDtypeStruct((_N, _D), jnp.float32),
    mesh=_MESH,
    scratch_types=[
        pltpu.VMEM((8, _CHUNK), jnp.int32),
        pltpu.VMEM((8, _CHUNK), jnp.int32),
        pltpu.VMEM((2, _CHUNK, _D), jnp.float32),
        pltpu.VMEM_SHARED((_ACC_ROWS, _D), jnp.float32),
        pltpu.SemaphoreType.DMA,
        pltpu.SemaphoreType.DMA,
    ],
    compiler_params=_SC_PARAMS)


def _count_body(dstr, z16, ones_h, cnt_hbm, idx_d, ones_v, cacc, csem):
  sid = lax.axis_index("s")

  z0 = sid * _ZROWS
  pltpu.sync_copy(z16.at[pl.ds(0, _ZROWS)], cacc.at[pl.ds(z0, _ZROWS)])
  pltpu.sync_copy(ones_h, ones_v)
  plsc.subcore_barrier()

  pltpu.sync_copy(dstr.at[pl.ds(sid * _RPW, _RPW)], idx_d)

  @pl.loop(0, _RPW // 8)
  def _(c):
    for j in range(8):
      pltpu.async_copy(ones_v, cacc.at[idx_d.at[c * 8 + j]], csem, add=True)
    for j in range(8):
      pltpu.make_async_copy(ones_v, cacc.at[idx_d.at[c * 8 + j]], csem).wait()

  plsc.subcore_barrier()
  w0 = sid * _WROWS

  @pl.when(sid < _NS - 1)
  def _():
    pltpu.sync_copy(cacc.at[pl.ds(w0, _WROWS)], cnt_hbm.at[pl.ds(w0, _WROWS)])

  @pl.when(sid == _NS - 1)
  def _():
    w1 = (_NS - 1) * _WROWS
    pltpu.sync_copy(cacc.at[pl.ds(w1, _WLAST)], cnt_hbm.at[pl.ds(w1, _WLAST)])


_sc_count = pl.kernel(
    _count_body,
    out_type=jax.ShapeDtypeStruct((_N, 16), jnp.float32),
    mesh=_MESH1,
    scratch_types=[
        pltpu.VMEM((_RPW, _CHUNK), jnp.int32),
        pltpu.VMEM((_CHUNK, 16), jnp.float32),
        pltpu.VMEM_SHARED((_ACC_ROWS, 16), jnp.float32),
        pltpu.SemaphoreType.DMA,
    ],
    compiler_params=_SC_PARAMS)


_R = 1000


def _self_mm_body(h_r, w_r, o_r):
  o_r[...] = jnp.dot(h_r[...], w_r[...], preferred_element_type=jnp.float32)


def _combine_body(p_r, s_r, c_r, wn_r, b_r, o_r):
  inv = 1.0 / jnp.maximum(c_r[:, 0:1], 1.0)
  mean = s_r[...] * inv
  h = (p_r[...]
       + jnp.dot(mean, wn_r[...], preferred_element_type=jnp.float32)
       + b_r[...])
  o_r[...] = jnp.maximum(h, 0.0)


def _combine2_body(p_r, s_r, c_r, wn_r, b_r, wc1_r, bc1_r, wc2_r, bc2_r, o_r):
  inv = 1.0 / jnp.maximum(c_r[:, 0:1], 1.0)
  mean = s_r[...] * inv
  h2 = (p_r[...]
        + jnp.dot(mean, wn_r[...], preferred_element_type=jnp.float32)
        + b_r[...])
  h2 = jnp.maximum(h2, 0.0)
  hid = jnp.maximum(
      jnp.dot(h2, wc1_r[...], preferred_element_type=jnp.float32) + bc1_r[...],
      0.0)
  o_r[...] = (jnp.dot(hid, wc2_r[...], preferred_element_type=jnp.float32)
              + bc2_r[...])


def _row_spec(w):
  return pl.BlockSpec((_R, w), lambda i: (i, 0))


def _full_spec(h, w):
  return pl.BlockSpec((h, w), lambda i: (0, 0))


def _self_mm(h, W):
  return pl.pallas_call(
      _self_mm_body,
      grid=(_N // _R,),
      in_specs=[_row_spec(_D), _full_spec(_D, _D)],
      out_specs=_row_spec(_D),
      out_shape=jax.ShapeDtypeStruct((_N, _D), jnp.float32),
  )(h, W)


def _combine1(p, sums, cnts, Wn, b):
  return pl.pallas_call(
      _combine_body,
      grid=(_N // _R,),
      in_specs=[
          _row_spec(_D), _row_spec(_D), _row_spec(16),
          _full_spec(_D, _D), _full_spec(1, _D),
      ],
      out_specs=_row_spec(_D),
      out_shape=jax.ShapeDtypeStruct((_N, _D), jnp.float32),
  )(p, sums, cnts, Wn, b.reshape(1, _D))


def _combine2(p, sums, cnts, Wn, b, Wc1, bc1, Wc2, bc2):
  n_cls = Wc2.shape[1]
  cls_hid = Wc1.shape[1]
  return pl.pallas_call(
      _combine2_body,
      grid=(_N // _R,),
      in_specs=[
          _row_spec(_D), _row_spec(_D), _row_spec(16),
          _full_spec(_D, _D), _full_spec(1, _D),
          _full_spec(_D, cls_hid), _full_spec(1, cls_hid),
          _full_spec(cls_hid, n_cls), _full_spec(1, n_cls),
      ],
      out_specs=_row_spec(n_cls),
      out_shape=jax.ShapeDtypeStruct((_N, n_cls), jnp.float32),
  )(p, sums, cnts, Wn, b.reshape(1, _D), Wc1, bc1.reshape(1, cls_hid),
    Wc2, bc2.reshape(1, n_cls))


def kernel(x, edge_index, W1_self, W1_neigh, b1, W2_self, W2_neigh, b2,
           Wc1, bc1, Wc2, bc2):
  pad = _EROWS_PAD * _CHUNK - _E
  src = jnp.concatenate([edge_index[0], jnp.zeros((pad,), jnp.int32)])
  dst_pad = _N + (jnp.arange(pad, dtype=jnp.int32) % (_ACC_ROWS - _N))
  dst = jnp.concatenate([edge_index[1], dst_pad])
  src = src.reshape(_EROWS_PAD, _CHUNK)
  dst = dst.reshape(_EROWS_PAD, _CHUNK)

  z16 = jnp.zeros((_ZROWS, 16), jnp.float32)
  ones16 = jnp.ones((_CHUNK, 16), jnp.float32)

  cnts = _sc_count(dst, z16, ones16)
  sums1 = _sc_agg(x, src, dst)
  p1 = _self_mm(x, W1_self)
  h1 = _combine1(p1, sums1, cnts, W1_neigh, b1)

  sums2 = _sc_agg(h1, src, dst)
  p2 = _self_mm(h1, W2_self)
  return _combine2(p2, sums2, cnts, W2_neigh, b2, Wc1, bc1, Wc2, bc2)


# speedup vs baseline: 2.2459x; 2.2256x over previous
import jax
import jax.numpy as jnp
from jax import lax
from jax.experimental import pallas as pl
from jax.experimental.pallas import tpu as pltpu
from jax.experimental.pallas import tpu_sc as plsc

_N = 10000
_E = 320000
_D = 128
_CHUNK = 128
_NC = 2
_NS = 16
_NW = _NC * _NS
_EROWS = -(-_E // _CHUNK)
_EROWS_PAD = -(-_EROWS // (_NW * 8)) * _NW * 8
_RPW = _EROWS_PAD // _NS
_ACC_ROWS = 10240
_ZROWS = _ACC_ROWS // _NS
_WROWS = 624
_WLAST = _N - 15 * _WROWS

_SC_PARAMS = pltpu.CompilerParams(use_tc_tiling_on_sc=False)
_MESH = plsc.VectorSubcoreMesh(core_axis_name="c", subcore_axis_name="s")
_MESH1 = plsc.VectorSubcoreMesh(core_axis_name="c", subcore_axis_name="s",
                                num_cores=1, num_subcores=_NS)


def _agg_body(h_hbm, srcr, dstr, sums_hbm, idx_s, idx_d, rows2, acc,
              sem0, sem1):
  cid = lax.axis_index("c")
  sid = lax.axis_index("s")
  sems = (sem0, sem1)

  @pl.when(cid == 0)
  def _():
    @pl.loop(0, _CHUNK)
    def _(r):
      for c in range(_D // 16):
        rows2[0, r, pl.ds(c * 16, 16)] = jnp.zeros((16,), jnp.float32)

    z0 = sid * _ZROWS
    for k in range(_ZROWS // _CHUNK):
      pltpu.sync_copy(rows2.at[0], acc.at[pl.ds(z0 + k * _CHUNK, _CHUNK)])

  plsc.subcore_barrier()

  def stage(r0, c):
    pltpu.sync_copy(srcr.at[pl.ds(r0 + c * 8, 8)], idx_s)
    pltpu.sync_copy(dstr.at[pl.ds(r0 + c * 8, 8)], idx_d)

  def wait_scatter(b):
    pltpu.make_async_copy(rows2.at[b], acc.at[idx_d.at[b]], sems[b]).wait()

  def row(j, wait):
    b = j % 2
    if wait:
      wait_scatter(b)
    pltpu.sync_copy(h_hbm.at[idx_s.at[j]], rows2.at[b])
    pltpu.async_copy(rows2.at[b], acc.at[idx_d.at[j]], sems[b], add=True)

  def pipeline(r0, nchunk):
    stage(r0, 0)
    for j in range(8):
      row(j, j >= 2)

    @pl.loop(1, nchunk)
    def _(c):
      wait_scatter(0)
      wait_scatter(1)
      stage(r0, c)
      for j in range(8):
        row(j, j >= 2)

    wait_scatter(0)
    wait_scatter(1)

  @pl.when(cid == 0)
  def _():
    pipeline(sid * _RPW, _RPW // 8)

  plsc.subcore_barrier()

  @pl.when(cid == 0)
  def _():
    w0 = sid * _WROWS

    @pl.when(sid < _NS - 1)
    def _():
      pltpu.sync_copy(acc.at[pl.ds(w0, _WROWS)], sums_hbm.at[pl.ds(w0, _WROWS)])

    @pl.when(sid == _NS - 1)
    def _():
      w1 = (_NS - 1) * _WROWS
      pltpu.sync_copy(acc.at[pl.ds(w1, _WLAST)], sums_hbm.at[pl.ds(w1, _WLAST)])


_sc_agg = pl.kernel(
    _agg_body,
    out_type=jax.ShapeDtypeStruct((_N, _D), jnp.float32),
    mesh=_MESH,
    scratch_types=[
        pltpu.VMEM((8, _CHUNK), jnp.int32),
        pltpu.VMEM((8, _CHUNK), jnp.int32),
        pltpu.VMEM((2, _CHUNK, _D), jnp.float32),
        pltpu.VMEM_SHARED((_ACC_ROWS, _D), jnp.float32),
        pltpu.SemaphoreType.DMA,
        pltpu.SemaphoreType.DMA,
    ],
    compiler_params=_SC_PARAMS)


def _count_body(dstr, z16, ones_h, cnt_hbm, idx_d, ones_v, cacc, csem):
  sid = lax.axis_index("s")

  z0 = sid * _ZROWS
  pltpu.sync_copy(z16.at[pl.ds(0, _ZROWS)], cacc.at[pl.ds(z0, _ZROWS)])
  pltpu.sync_copy(ones_h, ones_v)
  plsc.subcore_barrier()

  pltpu.sync_copy(dstr.at[pl.ds(sid * _RPW, _RPW)], idx_d)

  @pl.loop(0, _RPW // 8)
  def _(c):
    for j in range(8):
      pltpu.async_copy(ones_v, cacc.at[idx_d.at[c * 8 + j]], csem, add=True)
    for j in range(8):
      pltpu.make_async_copy(ones_v, cacc.at[idx_d.at[c * 8 + j]], csem).wait()

  plsc.subcore_barrier()
  w0 = sid * _WROWS

  @pl.when(sid < _NS - 1)
  def _():
    pltpu.sync_copy(cacc.at[pl.ds(w0, _WROWS)], cnt_hbm.at[pl.ds(w0, _WROWS)])

  @pl.when(sid == _NS - 1)
  def _():
    w1 = (_NS - 1) * _WROWS
    pltpu.sync_copy(cacc.at[pl.ds(w1, _WLAST)], cnt_hbm.at[pl.ds(w1, _WLAST)])


_sc_count = pl.kernel(
    _count_body,
    out_type=jax.ShapeDtypeStruct((_N, 16), jnp.float32),
    mesh=_MESH1,
    scratch_types=[
        pltpu.VMEM((_RPW, _CHUNK), jnp.int32),
        pltpu.VMEM((_CHUNK, 16), jnp.float32),
        pltpu.VMEM_SHARED((_ACC_ROWS, 16), jnp.float32),
        pltpu.SemaphoreType.DMA,
    ],
    compiler_params=_SC_PARAMS)


_R = 1000


def _self_mm_body(h_r, w_r, o_r):
  o_r[...] = jnp.dot(h_r[...], w_r[...], preferred_element_type=jnp.float32)


def _combine_body(p_r, s_r, c_r, wn_r, b_r, o_r):
  inv = 1.0 / jnp.maximum(c_r[:, 0:1], 1.0)
  mean = s_r[...] * inv
  h = (p_r[...]
       + jnp.dot(mean, wn_r[...], preferred_element_type=jnp.float32)
       + b_r[...])
  o_r[...] = jnp.maximum(h, 0.0)


def _combine2_body(p_r, s_r, c_r, wn_r, b_r, wc1_r, bc1_r, wc2_r, bc2_r, o_r):
  inv = 1.0 / jnp.maximum(c_r[:, 0:1], 1.0)
  mean = s_r[...] * inv
  h2 = (p_r[...]
        + jnp.dot(mean, wn_r[...], preferred_element_type=jnp.float32)
        + b_r[...])
  h2 = jnp.maximum(h2, 0.0)
  hid = jnp.maximum(
      jnp.dot(h2, wc1_r[...], preferred_element_type=jnp.float32) + bc1_r[...],
      0.0)
  o_r[...] = (jnp.dot(hid, wc2_r[...], preferred_element_type=jnp.float32)
              + bc2_r[...])


def _row_spec(w):
  return pl.BlockSpec((_R, w), lambda i: (i, 0))


def _full_spec(h, w):
  return pl.BlockSpec((h, w), lambda i: (0, 0))


def _self_mm(h, W):
  return pl.pallas_call(
      _self_mm_body,
      grid=(_N // _R,),
      in_specs=[_row_spec(_D), _full_spec(_D, _D)],
      out_specs=_row_spec(_D),
      out_shape=jax.ShapeDtypeStruct((_N, _D), jnp.float32),
  )(h, W)


def _combine1(p, sums, cnts, Wn, b):
  return pl.pallas_call(
      _combine_body,
      grid=(_N // _R,),
      in_specs=[
          _row_spec(_D), _row_spec(_D), _row_spec(16),
          _full_spec(_D, _D), _full_spec(1, _D),
      ],
      out_specs=_row_spec(_D),
      out_shape=jax.ShapeDtypeStruct((_N, _D), jnp.float32),
  )(p, sums, cnts, Wn, b.reshape(1, _D))


def _combine2(p, sums, cnts, Wn, b, Wc1, bc1, Wc2, bc2):
  n_cls = Wc2.shape[1]
  cls_hid = Wc1.shape[1]
  return pl.pallas_call(
      _combine2_body,
      grid=(_N // _R,),
      in_specs=[
          _row_spec(_D), _row_spec(_D), _row_spec(16),
          _full_spec(_D, _D), _full_spec(1, _D),
          _full_spec(_D, cls_hid), _full_spec(1, cls_hid),
          _full_spec(cls_hid, n_cls), _full_spec(1, n_cls),
      ],
      out_specs=_row_spec(n_cls),
      out_shape=jax.ShapeDtypeStruct((_N, n_cls), jnp.float32),
  )(p, sums, cnts, Wn, b.reshape(1, _D), Wc1, bc1.reshape(1, cls_hid),
    Wc2, bc2.reshape(1, n_cls))


def kernel(x, edge_index, W1_self, W1_neigh, b1, W2_self, W2_neigh, b2,
           Wc1, bc1, Wc2, bc2):
  pad = _EROWS_PAD * _CHUNK - _E
  ar = jnp.arange(pad, dtype=jnp.int32)
  src = jnp.concatenate([edge_index[0], ar % _N])
  dst = jnp.concatenate([edge_index[1], _N + ar % (_ACC_ROWS - _N)])
  src = src.reshape(_EROWS_PAD, _CHUNK)
  dst = dst.reshape(_EROWS_PAD, _CHUNK)

  z16 = jnp.zeros((_ZROWS, 16), jnp.float32)
  ones16 = jnp.ones((_CHUNK, 16), jnp.float32)

  cnts = _sc_count(dst, z16, ones16)
  sums1 = _sc_agg(x, src, dst)
  p1 = _self_mm(x, W1_self)
  h1 = _combine1(p1, sums1, cnts, W1_neigh, b1)

  sums2 = _sc_agg(h1, src, dst)
  p2 = _self_mm(h1, W2_self)
  return _combine2(p2, sums2, cnts, W2_neigh, b2, Wc1, bc1, Wc2, bc2)


# speedup vs baseline: 3.7172x; 1.6551x over previous
import jax
import jax.numpy as jnp
from jax import lax
from jax.experimental import pallas as pl
from jax.experimental.pallas import tpu as pltpu
from jax.experimental.pallas import tpu_sc as plsc

_N = 10000
_E = 320000
_D = 128
_CHUNK = 128
_NC = 2
_NS = 16
_NW = _NC * _NS
_EROWS = -(-_E // _CHUNK)
_EROWS_PAD = -(-_EROWS // (_NW * 8)) * _NW * 8
_RPW = _EROWS_PAD // _NS
_RPWW = _EROWS_PAD // _NW
_ACC_ROWS = 10240
_ZROWS = _ACC_ROWS // _NS
_WROWS = 624
_WLAST = _N - 15 * _WROWS

_SC_PARAMS = pltpu.CompilerParams(use_tc_tiling_on_sc=False)
_MESH = plsc.VectorSubcoreMesh(core_axis_name="c", subcore_axis_name="s")
_MESH1 = plsc.VectorSubcoreMesh(core_axis_name="c", subcore_axis_name="s",
                                num_cores=1, num_subcores=_NS)


def _agg_body(h_hbm, srcr, dstr, sums_hbm, idx_s, idx_d, rows2, acc,
              sem0, sem1):
  cid = lax.axis_index("c")
  sid = lax.axis_index("s")
  sems = (sem0, sem1)

  @pl.loop(0, _CHUNK)
  def _(r):
    for c in range(_D // 16):
      rows2[0, r, pl.ds(c * 16, 16)] = jnp.zeros((16,), jnp.float32)

  z0 = sid * _ZROWS
  for k in range(_ZROWS // _CHUNK):
    pltpu.sync_copy(rows2.at[0], acc.at[pl.ds(z0 + k * _CHUNK, _CHUNK)])

  plsc.subcore_barrier()

  def stage(r0, c):
    pltpu.sync_copy(srcr.at[pl.ds(r0 + c * 8, 8)], idx_s)
    pltpu.sync_copy(dstr.at[pl.ds(r0 + c * 8, 8)], idx_d)

  def wait_scatter(b):
    pltpu.make_async_copy(rows2.at[b], acc.at[idx_d.at[b]], sems[b]).wait()

  def row(j, wait):
    b = j % 2
    if wait:
      wait_scatter(b)
    pltpu.sync_copy(h_hbm.at[idx_s.at[j]], rows2.at[b])
    pltpu.async_copy(rows2.at[b], acc.at[idx_d.at[j]], sems[b], add=True)

  def pipeline(r0, nchunk):
    stage(r0, 0)
    for j in range(8):
      row(j, j >= 2)

    @pl.loop(1, nchunk)
    def _(c):
      wait_scatter(0)
      wait_scatter(1)
      stage(r0, c)
      for j in range(8):
        row(j, j >= 2)

    wait_scatter(0)
    wait_scatter(1)

  pipeline((cid * _NS + sid) * _RPWW, _RPWW // 8)

  plsc.subcore_barrier()

  w0 = sid * _WROWS

  @pl.when(sid < _NS - 1)
  def _():
    pltpu.sync_copy(acc.at[pl.ds(w0, _WROWS)],
                    sums_hbm.at[cid].at[pl.ds(w0, _WROWS)])

  @pl.when(sid == _NS - 1)
  def _():
    w1 = (_NS - 1) * _WROWS
    pltpu.sync_copy(acc.at[pl.ds(w1, _WLAST)],
                    sums_hbm.at[cid].at[pl.ds(w1, _WLAST)])


_sc_agg = pl.kernel(
    _agg_body,
    out_type=jax.ShapeDtypeStruct((_NC, _N, _D), jnp.float32),
    mesh=_MESH,
    scratch_types=[
        pltpu.VMEM((8, _CHUNK), jnp.int32),
        pltpu.VMEM((8, _CHUNK), jnp.int32),
        pltpu.VMEM((2, _CHUNK, _D), jnp.float32),
        pltpu.VMEM_SHARED((_ACC_ROWS, _D), jnp.float32),
        pltpu.SemaphoreType.DMA,
        pltpu.SemaphoreType.DMA,
    ],
    compiler_params=_SC_PARAMS)


def _count_body(dstr, z16, ones_h, cnt_hbm, idx_d, ones_v, cacc, csem):
  sid = lax.axis_index("s")

  z0 = sid * _ZROWS
  pltpu.sync_copy(z16.at[pl.ds(0, _ZROWS)], cacc.at[pl.ds(z0, _ZROWS)])
  pltpu.sync_copy(ones_h, ones_v)
  plsc.subcore_barrier()

  pltpu.sync_copy(dstr.at[pl.ds(sid * _RPW, _RPW)], idx_d)

  @pl.loop(0, _RPW // 8)
  def _(c):
    for j in range(8):
      pltpu.async_copy(ones_v, cacc.at[idx_d.at[c * 8 + j]], csem, add=True)
    for j in range(8):
      pltpu.make_async_copy(ones_v, cacc.at[idx_d.at[c * 8 + j]], csem).wait()

  plsc.subcore_barrier()
  w0 = sid * _WROWS

  @pl.when(sid < _NS - 1)
  def _():
    pltpu.sync_copy(cacc.at[pl.ds(w0, _WROWS)], cnt_hbm.at[pl.ds(w0, _WROWS)])

  @pl.when(sid == _NS - 1)
  def _():
    w1 = (_NS - 1) * _WROWS
    pltpu.sync_copy(cacc.at[pl.ds(w1, _WLAST)], cnt_hbm.at[pl.ds(w1, _WLAST)])


_sc_count = pl.kernel(
    _count_body,
    out_type=jax.ShapeDtypeStruct((_N, 16), jnp.float32),
    mesh=_MESH1,
    scratch_types=[
        pltpu.VMEM((_RPW, _CHUNK), jnp.int32),
        pltpu.VMEM((_CHUNK, 16), jnp.float32),
        pltpu.VMEM_SHARED((_ACC_ROWS, 16), jnp.float32),
        pltpu.SemaphoreType.DMA,
    ],
    compiler_params=_SC_PARAMS)


_R = 1000


def _self_mm_body(h_r, w_r, o_r):
  o_r[...] = jnp.dot(h_r[...], w_r[...], preferred_element_type=jnp.float32)


def _combine_body(p_r, s_r, c_r, wn_r, b_r, o_r):
  inv = 1.0 / jnp.maximum(c_r[:, 0:1], 1.0)
  mean = (s_r[0] + s_r[1]) * inv
  h = (p_r[...]
       + jnp.dot(mean, wn_r[...], preferred_element_type=jnp.float32)
       + b_r[...])
  o_r[...] = jnp.maximum(h, 0.0)


def _combine2_body(p_r, s_r, c_r, wn_r, b_r, wc1_r, bc1_r, wc2_r, bc2_r, o_r):
  inv = 1.0 / jnp.maximum(c_r[:, 0:1], 1.0)
  mean = (s_r[0] + s_r[1]) * inv
  h2 = (p_r[...]
        + jnp.dot(mean, wn_r[...], preferred_element_type=jnp.float32)
        + b_r[...])
  h2 = jnp.maximum(h2, 0.0)
  hid = jnp.maximum(
      jnp.dot(h2, wc1_r[...], preferred_element_type=jnp.float32) + bc1_r[...],
      0.0)
  o_r[...] = (jnp.dot(hid, wc2_r[...], preferred_element_type=jnp.float32)
              + bc2_r[...])


def _row_spec(w):
  return pl.BlockSpec((_R, w), lambda i: (i, 0))


def _part_spec(w):
  return pl.BlockSpec((_NC, _R, w), lambda i: (0, i, 0))


def _full_spec(h, w):
  return pl.BlockSpec((h, w), lambda i: (0, 0))


def _self_mm(h, W):
  return pl.pallas_call(
      _self_mm_body,
      grid=(_N // _R,),
      in_specs=[_row_spec(_D), _full_spec(_D, _D)],
      out_specs=_row_spec(_D),
      out_shape=jax.ShapeDtypeStruct((_N, _D), jnp.float32),
  )(h, W)


def _combine1(p, sums, cnts, Wn, b):
  return pl.pallas_call(
      _combine_body,
      grid=(_N // _R,),
      in_specs=[
          _row_spec(_D), _part_spec(_D), _row_spec(16),
          _full_spec(_D, _D), _full_spec(1, _D),
      ],
      out_specs=_row_spec(_D),
      out_shape=jax.ShapeDtypeStruct((_N, _D), jnp.float32),
  )(p, sums, cnts, Wn, b.reshape(1, _D))


def _combine2(p, sums, cnts, Wn, b, Wc1, bc1, Wc2, bc2):
  n_cls = Wc2.shape[1]
  cls_hid = Wc1.shape[1]
  return pl.pallas_call(
      _combine2_body,
      grid=(_N // _R,),
      in_specs=[
          _row_spec(_D), _part_spec(_D), _row_spec(16),
          _full_spec(_D, _D), _full_spec(1, _D),
          _full_spec(_D, cls_hid), _full_spec(1, cls_hid),
          _full_spec(cls_hid, n_cls), _full_spec(1, n_cls),
      ],
      out_specs=_row_spec(n_cls),
      out_shape=jax.ShapeDtypeStruct((_N, n_cls), jnp.float32),
  )(p, sums, cnts, Wn, b.reshape(1, _D), Wc1, bc1.reshape(1, cls_hid),
    Wc2, bc2.reshape(1, n_cls))


def kernel(x, edge_index, W1_self, W1_neigh, b1, W2_self, W2_neigh, b2,
           Wc1, bc1, Wc2, bc2):
  pad = _EROWS_PAD * _CHUNK - _E
  ar = jnp.arange(pad, dtype=jnp.int32)
  src = jnp.concatenate([edge_index[0], ar % _N])
  dst = jnp.concatenate([edge_index[1], _N + ar % (_ACC_ROWS - _N)])
  src = src.reshape(_EROWS_PAD, _CHUNK)
  dst = dst.reshape(_EROWS_PAD, _CHUNK)

  z16 = jnp.zeros((_ZROWS, 16), jnp.float32)
  ones16 = jnp.ones((_CHUNK, 16), jnp.float32)

  cnts = _sc_count(dst, z16, ones16)
  sums1 = _sc_agg(x, src, dst)
  p1 = _self_mm(x, W1_self)
  h1 = _combine1(p1, sums1, cnts, W1_neigh, b1)

  sums2 = _sc_agg(h1, src, dst)
  p2 = _self_mm(h1, W2_self)
  return _combine2(p2, sums2, cnts, W2_neigh, b2, Wc1, bc1, Wc2, bc2)


# speedup vs baseline: 4.1582x; 1.1186x over previous
import jax
import jax.numpy as jnp
from jax import lax
from jax.experimental import pallas as pl
from jax.experimental.pallas import tpu as pltpu
from jax.experimental.pallas import tpu_sc as plsc

_N = 10000
_E = 320000
_D = 128
_CHUNK = 128
_NC = 2
_NS = 16
_NW = _NC * _NS
_EROWS = -(-_E // _CHUNK)
_EROWS_PAD = -(-_EROWS // (_NW * 8)) * _NW * 8
_RPW = _EROWS_PAD // _NS
_GCH = 64
_GROWS = _EROWS_PAD * _CHUNK // _GCH
_G_RPW = _GROWS // _NW
_NGRP = _G_RPW // 4
_ACC_ROWS = 10240
_ZROWS = _ACC_ROWS // _NS
_WROWS = 624
_WLAST = _N - 15 * _WROWS

_SC_PARAMS = pltpu.CompilerParams(use_tc_tiling_on_sc=False)
_MESH = plsc.VectorSubcoreMesh(core_axis_name="c", subcore_axis_name="s")
_MESH1 = plsc.VectorSubcoreMesh(core_axis_name="c", subcore_axis_name="s",
                                num_cores=1, num_subcores=_NS)


def _agg_body(h_hbm, srcr, dstr, sums_hbm, idx_s, idx_d, rows4, acc,
              gs0, gs1, gs2, gs3, ss0, ss1, ss2, ss3):
  cid = lax.axis_index("c")
  sid = lax.axis_index("s")
  gsems = (gs0, gs1, gs2, gs3)
  ssems = (ss0, ss1, ss2, ss3)

  @pl.loop(0, _GCH)
  def _(r):
    for c in range(_D // 16):
      rows4[0, r, pl.ds(c * 16, 16)] = jnp.zeros((16,), jnp.float32)

  z0 = sid * _ZROWS
  for k in range(_ZROWS // _GCH):
    pltpu.sync_copy(rows4.at[0], acc.at[pl.ds(z0 + k * _GCH, _GCH)])

  plsc.subcore_barrier()

  gr0 = (cid * _NS + sid) * _G_RPW

  def stage(m):
    s = (m % 2) * 4
    pltpu.sync_copy(srcr.at[pl.ds(gr0 + m * 4, 4)], idx_s.at[pl.ds(s, 4)])
    pltpu.sync_copy(dstr.at[pl.ds(gr0 + m * 4, 4)], idx_d.at[pl.ds(s, 4)])

  def start_g(b, pos):
    pltpu.async_copy(h_hbm.at[idx_s.at[pos]], rows4.at[b], gsems[b])

  def wait_g(b):
    pltpu.make_async_copy(h_hbm.at[idx_s.at[0]], rows4.at[b], gsems[b]).wait()

  def start_s(b, pos):
    pltpu.async_copy(rows4.at[b], acc.at[idx_d.at[pos]], ssems[b], add=True)

  def wait_s(b):
    pltpu.make_async_copy(rows4.at[b], acc.at[idx_d.at[0]], ssems[b]).wait()

  def rowstep(i, m, wait_scatter, gather_ahead):
    pos = (m % 2) * 4 + i
    wait_g(i)
    start_s(i, pos)
    if wait_scatter:
      wait_s((i + 2) % 4)
    if gather_ahead:
      if i < 2:
        start_g((i + 2) % 4, (m % 2) * 4 + i + 2)
      else:
        start_g((i + 2) % 4, ((m + 1) % 2) * 4 + (i - 2))

  stage(0)
  start_g(0, 0)
  start_g(1, 1)
  rowstep(0, 0, False, True)
  rowstep(1, 0, False, True)
  stage(1)
  rowstep(2, 0, True, True)
  rowstep(3, 0, True, True)

  @pl.loop(1, _NGRP - 1)
  def _(m):
    rowstep(0, m, True, True)
    rowstep(1, m, True, True)
    stage(m + 1)
    rowstep(2, m, True, True)
    rowstep(3, m, True, True)

  mL = _NGRP - 1
  rowstep(0, mL, True, True)
  rowstep(1, mL, True, True)
  rowstep(2, mL, True, False)
  rowstep(3, mL, True, False)
  wait_s(2)
  wait_s(3)

  plsc.subcore_barrier()

  w0 = sid * _WROWS

  @pl.when(sid < _NS - 1)
  def _():
    pltpu.sync_copy(acc.at[pl.ds(w0, _WROWS)],
                    sums_hbm.at[cid].at[pl.ds(w0, _WROWS)])

  @pl.when(sid == _NS - 1)
  def _():
    w1 = (_NS - 1) * _WROWS
    pltpu.sync_copy(acc.at[pl.ds(w1, _WLAST)],
                    sums_hbm.at[cid].at[pl.ds(w1, _WLAST)])


_sc_agg = pl.kernel(
    _agg_body,
    out_type=jax.ShapeDtypeStruct((_NC, _N, _D), jnp.float32),
    mesh=_MESH,
    scratch_types=[
        pltpu.VMEM((8, _GCH), jnp.int32),
        pltpu.VMEM((8, _GCH), jnp.int32),
        pltpu.VMEM((4, _GCH, _D), jnp.float32),
        pltpu.VMEM_SHARED((_ACC_ROWS, _D), jnp.float32),
        pltpu.SemaphoreType.DMA, pltpu.SemaphoreType.DMA,
        pltpu.SemaphoreType.DMA, pltpu.SemaphoreType.DMA,
        pltpu.SemaphoreType.DMA, pltpu.SemaphoreType.DMA,
        pltpu.SemaphoreType.DMA, pltpu.SemaphoreType.DMA,
    ],
    compiler_params=_SC_PARAMS)


def _count_body(dstr, z16, ones_h, cnt_hbm, idx_d, ones_v, cacc, csem):
  sid = lax.axis_index("s")

  z0 = sid * _ZROWS
  pltpu.sync_copy(z16.at[pl.ds(0, _ZROWS)], cacc.at[pl.ds(z0, _ZROWS)])
  pltpu.sync_copy(ones_h, ones_v)
  plsc.subcore_barrier()

  pltpu.sync_copy(dstr.at[pl.ds(sid * _RPW, _RPW)], idx_d)

  @pl.loop(0, _RPW // 8)
  def _(c):
    for j in range(8):
      pltpu.async_copy(ones_v, cacc.at[idx_d.at[c * 8 + j]], csem, add=True)
    for j in range(8):
      pltpu.make_async_copy(ones_v, cacc.at[idx_d.at[c * 8 + j]], csem).wait()

  plsc.subcore_barrier()
  w0 = sid * _WROWS

  @pl.when(sid < _NS - 1)
  def _():
    pltpu.sync_copy(cacc.at[pl.ds(w0, _WROWS)], cnt_hbm.at[pl.ds(w0, _WROWS)])

  @pl.when(sid == _NS - 1)
  def _():
    w1 = (_NS - 1) * _WROWS
    pltpu.sync_copy(cacc.at[pl.ds(w1, _WLAST)], cnt_hbm.at[pl.ds(w1, _WLAST)])


_sc_count = pl.kernel(
    _count_body,
    out_type=jax.ShapeDtypeStruct((_N, 16), jnp.float32),
    mesh=_MESH1,
    scratch_types=[
        pltpu.VMEM((_RPW, _CHUNK), jnp.int32),
        pltpu.VMEM((_CHUNK, 16), jnp.float32),
        pltpu.VMEM_SHARED((_ACC_ROWS, 16), jnp.float32),
        pltpu.SemaphoreType.DMA,
    ],
    compiler_params=_SC_PARAMS)


_R = 1000


def _self_mm_body(h_r, w_r, o_r):
  o_r[...] = jnp.dot(h_r[...], w_r[...], preferred_element_type=jnp.float32)


def _combine_body(p_r, s_r, c_r, wn_r, b_r, o_r):
  inv = 1.0 / jnp.maximum(c_r[:, 0:1], 1.0)
  mean = (s_r[0] + s_r[1]) * inv
  h = (p_r[...]
       + jnp.dot(mean, wn_r[...], preferred_element_type=jnp.float32)
       + b_r[...])
  o_r[...] = jnp.maximum(h, 0.0)


def _combine2_body(p_r, s_r, c_r, wn_r, b_r, wc1_r, bc1_r, wc2_r, bc2_r, o_r):
  inv = 1.0 / jnp.maximum(c_r[:, 0:1], 1.0)
  mean = (s_r[0] + s_r[1]) * inv
  h2 = (p_r[...]
        + jnp.dot(mean, wn_r[...], preferred_element_type=jnp.float32)
        + b_r[...])
  h2 = jnp.maximum(h2, 0.0)
  hid = jnp.maximum(
      jnp.dot(h2, wc1_r[...], preferred_element_type=jnp.float32) + bc1_r[...],
      0.0)
  o_r[...] = (jnp.dot(hid, wc2_r[...], preferred_element_type=jnp.float32)
              + bc2_r[...])


def _row_spec(w):
  return pl.BlockSpec((_R, w), lambda i: (i, 0))


def _part_spec(w):
  return pl.BlockSpec((_NC, _R, w), lambda i: (0, i, 0))


def _full_spec(h, w):
  return pl.BlockSpec((h, w), lambda i: (0, 0))


def _self_mm(h, W):
  return pl.pallas_call(
      _self_mm_body,
      grid=(_N // _R,),
      in_specs=[_row_spec(_D), _full_spec(_D, _D)],
      out_specs=_row_spec(_D),
      out_shape=jax.ShapeDtypeStruct((_N, _D), jnp.float32),
  )(h, W)


def _combine1(p, sums, cnts, Wn, b):
  return pl.pallas_call(
      _combine_body,
      grid=(_N // _R,),
      in_specs=[
          _row_spec(_D), _part_spec(_D), _row_spec(16),
          _full_spec(_D, _D), _full_spec(1, _D),
      ],
      out_specs=_row_spec(_D),
      out_shape=jax.ShapeDtypeStruct((_N, _D), jnp.float32),
  )(p, sums, cnts, Wn, b.reshape(1, _D))


def _combine2(p, sums, cnts, Wn, b, Wc1, bc1, Wc2, bc2):
  n_cls = Wc2.shape[1]
  cls_hid = Wc1.shape[1]
  return pl.pallas_call(
      _combine2_body,
      grid=(_N // _R,),
      in_specs=[
          _row_spec(_D), _part_spec(_D), _row_spec(16),
          _full_spec(_D, _D), _full_spec(1, _D),
          _full_spec(_D, cls_hid), _full_spec(1, cls_hid),
          _full_spec(cls_hid, n_cls), _full_spec(1, n_cls),
      ],
      out_specs=_row_spec(n_cls),
      out_shape=jax.ShapeDtypeStruct((_N, n_cls), jnp.float32),
  )(p, sums, cnts, Wn, b.reshape(1, _D), Wc1, bc1.reshape(1, cls_hid),
    Wc2, bc2.reshape(1, n_cls))


def kernel(x, edge_index, W1_self, W1_neigh, b1, W2_self, W2_neigh, b2,
           Wc1, bc1, Wc2, bc2):
  pad = _EROWS_PAD * _CHUNK - _E
  ar = jnp.arange(pad, dtype=jnp.int32)
  srcf = jnp.concatenate([edge_index[0], ar % _N])
  dstf = jnp.concatenate([edge_index[1], _N + ar % (_ACC_ROWS - _N)])
  src = srcf.reshape(_GROWS, _GCH)
  dst = dstf.reshape(_GROWS, _GCH)
  dst128 = dstf.reshape(_EROWS_PAD, _CHUNK)

  z16 = jnp.zeros((_ZROWS, 16), jnp.float32)
  ones16 = jnp.ones((_CHUNK, 16), jnp.float32)

  cnts = _sc_count(dst128, z16, ones16)
  sums1 = _sc_agg(x, src, dst)
  p1 = _self_mm(x, W1_self)
  h1 = _combine1(p1, sums1, cnts, W1_neigh, b1)

  sums2 = _sc_agg(h1, src, dst)
  p2 = _self_mm(h1, W2_self)
  return _combine2(p2, sums2, cnts, W2_neigh, b2, Wc1, bc1, Wc2, bc2)
